# Initial kernel scaffold; baseline (speedup 1.0000x reference)
import jax
import jax.numpy as jnp
from jax.experimental import pallas as pl


def kernel(x, edge_index, W):
    raise NotImplementedError("write your pallas kernel here")


# speedup vs baseline: 6.9479x; 6.9479x over previous
import functools

import jax
import jax.numpy as jnp
from jax import lax
from jax.experimental import pallas as pl
from jax.experimental.pallas import tpu as pltpu
from jax.experimental.pallas import tpu_sc as plsc

H = 128
V = 10000
VP = 10240
E = 320000
AUG = 144
NS = 16
GW = 80
NG = E // GW
GPT = NG // NS
NB = 2
NBATCH = GPT // NB
RPT = VP // NS
ZR = NB * GW


def _sc_body(xaug, src_g, dst_g, outp, outq, acc, sidx, gidx, rows, gsem):
    c = lax.axis_index("c")
    s = lax.axis_index("s")
    base = s * RPT

    def zrow(i, carry):
        for j in range(AUG // 16):
            rows[i, pl.ds(j * 16, 16)] = jnp.zeros((16,), jnp.float32)
        return carry

    lax.fori_loop(0, ZR, zrow, 0)
    for k in range(RPT // ZR):
        pltpu.sync_copy(rows, acc.at[pl.ds(base + k * ZR, ZR)])
    plsc.subcore_barrier()

    def process(seg_hbm, gat_hbm, out_hbm):
        g0 = s * GPT

        def batch(b, carry):
            gb = g0 + b * NB
            pltpu.sync_copy(seg_hbm.at[pl.ds(gb, NB)], sidx)
            pltpu.sync_copy(gat_hbm.at[pl.ds(gb, NB)], gidx)
            descs = [
                pltpu.async_copy(
                    xaug.at[gidx.at[j]], rows.at[pl.ds(j * GW, GW)], gsem)
                for j in range(NB)
            ]
            for d in descs:
                d.wait()
            for j in range(NB):
                pltpu.sync_copy(
                    rows.at[pl.ds(j * GW, GW)], acc.at[sidx.at[j]], add=True)
            return carry

        lax.fori_loop(0, NBATCH, batch, 0)
        plsc.subcore_barrier()
        pltpu.sync_copy(acc.at[pl.ds(base, RPT)], out_hbm.at[pl.ds(base, RPT)])

    @pl.when(c == 0)
    def _():
        process(src_g, dst_g, outp)

    @pl.when(c == 1)
    def _():
        process(dst_g, src_g, outq)


_sc_accumulate = functools.partial(
    pl.kernel,
    out_type=(
        jax.ShapeDtypeStruct((VP, AUG), jnp.float32),
        jax.ShapeDtypeStruct((VP, AUG), jnp.float32),
    ),
    mesh=plsc.VectorSubcoreMesh(core_axis_name="c", subcore_axis_name="s"),
    compiler_params=pltpu.CompilerParams(use_tc_tiling_on_sc=False),
    scratch_types=[
        pltpu.VMEM_SHARED((VP, AUG), jnp.float32),
        pltpu.VMEM((NB, GW), jnp.int32),
        pltpu.VMEM((NB, GW), jnp.int32),
        pltpu.VMEM((ZR, AUG), jnp.float32),
        pltpu.SemaphoreType.DMA,
    ],
)(_sc_body)


def _mm_body(x_ref, p_ref, q_ref, w_ref, o_ref):
    xb = x_ref[...]
    pb = p_ref[...]
    qb = q_ref[...]
    w = w_ref[...]
    dout = pb[:, H:H + 1]
    din = qb[:, H:H + 1]
    dn = (((1,), (1,)), ((), ()))
    o_ref[...] = (
        lax.dot_general(xb * dout, w[:H, :H], dn, preferred_element_type=jnp.float32)
        + lax.dot_general(pb[:, :H], w[:H, H:], dn, preferred_element_type=jnp.float32)
        + lax.dot_general(qb[:, :H], w[H:, :H], dn, preferred_element_type=jnp.float32)
        + lax.dot_general(xb * din, w[H:, H:], dn, preferred_element_type=jnp.float32)
    )


_BM = 1000
_mm = pl.pallas_call(
    _mm_body,
    grid=(V // _BM,),
    in_specs=[
        pl.BlockSpec((_BM, H), lambda i: (i, 0)),
        pl.BlockSpec((_BM, AUG), lambda i: (i, 0)),
        pl.BlockSpec((_BM, AUG), lambda i: (i, 0)),
        pl.BlockSpec((2 * H, 2 * H), lambda i: (0, 0)),
    ],
    out_specs=pl.BlockSpec((_BM, H), lambda i: (i, 0)),
    out_shape=jax.ShapeDtypeStruct((V, H), jnp.float32),
)


def kernel(x, edge_index, W):
    src = edge_index[0].reshape(NG, GW)
    dst = edge_index[1].reshape(NG, GW)
    xaug = jnp.concatenate(
        [x, jnp.ones((V, 1), jnp.float32), jnp.zeros((V, AUG - H - 1), jnp.float32)],
        axis=1,
    )
    pa, qa = _sc_accumulate(xaug, src, dst)
    return _mm(x, pa[:V], qa[:V], W)


# speedup vs baseline: 9.2301x; 1.3285x over previous
import functools

import jax
import jax.numpy as jnp
from jax import lax
from jax.experimental import pallas as pl
from jax.experimental.pallas import tpu as pltpu
from jax.experimental.pallas import tpu_sc as plsc

H = 128
V = 10000
VP = 10240
E = 320000
AUG = 144
NS = 16
GW = 80
NG = E // GW
GPT = NG // NS
IB = 25
NCH = GPT // IB
RPT = VP // NS
ZR = 2 * GW


def _sc_body(xaug, idx2, out2, acc, ibuf, rows, gsem, ssem, isem):
    c = lax.axis_index("c")
    s = lax.axis_index("s")
    gsel = 1 - c
    base = s * RPT
    g0 = s * GPT

    def zrow(i, carry):
        for j in range(AUG // 16):
            rows[i, pl.ds(j * 16, 16)] = jnp.zeros((16,), jnp.float32)
        return carry

    lax.fori_loop(0, ZR, zrow, 0)
    for k in range(RPT // ZR):
        pltpu.sync_copy(rows, acc.at[pl.ds(base + k * ZR, ZR)])
    plsc.subcore_barrier()

    def gather_desc(buf, j, slot):
        return pltpu.make_async_copy(
            xaug.at[ibuf.at[buf, j, gsel]], rows.at[pl.ds(slot, GW)], gsem)

    def scatter_desc(buf, j, slot):
        return pltpu.make_async_copy(
            rows.at[pl.ds(slot, GW)], acc.at[ibuf.at[buf, j, c]], ssem)

    pltpu.sync_copy(idx2.at[pl.ds(g0, IB)], ibuf.at[0])
    pltpu.async_copy(xaug.at[ibuf.at[0, 0, gsel]], rows.at[pl.ds(0, GW)], gsem)

    def step(g, carry):
        ci = g // IB
        j = g - ci * IB
        buf = lax.rem(ci, 2)
        slot = lax.rem(g, 2) * GW
        gather_desc(buf, j, slot).wait()
        pltpu.async_copy(
            rows.at[pl.ds(slot, GW)], acc.at[ibuf.at[buf, j, c]], ssem,
            add=True)
        @pl.when(g >= 1)
        def _():
            g1 = g - 1
            ci1 = g1 // IB
            scatter_desc(lax.rem(ci1, 2), g1 - ci1 * IB,
                         lax.rem(g1, 2) * GW).wait()

        @pl.when(jnp.logical_and(j == 0, ci + 1 < NCH))
        def _():
            pltpu.async_copy(
                idx2.at[pl.ds(g0 + (ci + 1) * IB, IB)],
                ibuf.at[1 - buf], isem)

        @pl.when(g + 1 < GPT)
        def _():
            g2 = g + 1
            ci2 = g2 // IB
            j2 = g2 - ci2 * IB
            buf2 = lax.rem(ci2, 2)

            @pl.when(j2 == 0)
            def _():
                pltpu.make_async_copy(
                    idx2.at[pl.ds(g0 + ci2 * IB, IB)], ibuf.at[buf2],
                    isem).wait()

            pltpu.async_copy(
                xaug.at[ibuf.at[buf2, j2, gsel]],
                rows.at[pl.ds(lax.rem(g2, 2) * GW, GW)], gsem)

        return carry

    lax.fori_loop(0, GPT, step, 0)
    scatter_desc((NCH - 1) % 2, IB - 1, ((GPT - 1) % 2) * GW).wait()
    plsc.subcore_barrier()
    pltpu.sync_copy(acc.at[pl.ds(base, RPT)],
                    out2.at[c, pl.ds(base, RPT)])


_sc_accumulate = functools.partial(
    pl.kernel,
    out_type=jax.ShapeDtypeStruct((2, VP, AUG), jnp.float32),
    mesh=plsc.VectorSubcoreMesh(core_axis_name="c", subcore_axis_name="s"),
    compiler_params=pltpu.CompilerParams(use_tc_tiling_on_sc=False),
    scratch_types=[
        pltpu.VMEM_SHARED((VP, AUG), jnp.float32),
        pltpu.VMEM((2, IB, 2, GW), jnp.int32),
        pltpu.VMEM((ZR, AUG), jnp.float32),
        pltpu.SemaphoreType.DMA,
        pltpu.SemaphoreType.DMA,
        pltpu.SemaphoreType.DMA,
    ],
)(_sc_body)


def _mm_body(x_ref, p_ref, q_ref, w_ref, o_ref):
    xb = x_ref[...]
    pb = p_ref[0]
    qb = q_ref[0]
    w = w_ref[...]
    dout = pb[:, H:H + 1]
    din = qb[:, H:H + 1]
    dn = (((1,), (1,)), ((), ()))
    o_ref[...] = (
        lax.dot_general(xb * dout, w[:H, :H], dn, preferred_element_type=jnp.float32)
        + lax.dot_general(pb[:, :H], w[:H, H:], dn, preferred_element_type=jnp.float32)
        + lax.dot_general(qb[:, :H], w[H:, :H], dn, preferred_element_type=jnp.float32)
        + lax.dot_general(xb * din, w[H:, H:], dn, preferred_element_type=jnp.float32)
    )


_BM = 1000
_mm = pl.pallas_call(
    _mm_body,
    grid=(V // _BM,),
    in_specs=[
        pl.BlockSpec((_BM, H), lambda i: (i, 0)),
        pl.BlockSpec((1, _BM, AUG), lambda i: (0, i, 0)),
        pl.BlockSpec((1, _BM, AUG), lambda i: (1, i, 0)),
        pl.BlockSpec((2 * H, 2 * H), lambda i: (0, 0)),
    ],
    out_specs=pl.BlockSpec((_BM, H), lambda i: (i, 0)),
    out_shape=jax.ShapeDtypeStruct((V, H), jnp.float32),
)


def kernel(x, edge_index, W):
    idx2 = edge_index.reshape(2, NG, GW).transpose(1, 0, 2)
    xaug = jnp.concatenate(
        [x, jnp.ones((V, 1), jnp.float32), jnp.zeros((V, AUG - H - 1), jnp.float32)],
        axis=1,
    )
    pq = _sc_accumulate(xaug, idx2)
    return _mm(x, pq, pq, W)


# speedup vs baseline: 12.8492x; 1.3921x over previous
import functools

import jax
import jax.numpy as jnp
from jax import lax
from jax.experimental import pallas as pl
from jax.experimental.pallas import tpu as pltpu
from jax.experimental.pallas import tpu_sc as plsc

H = 128
V = 10000
VP = 10240
E = 320000
AUG = 144
NS = 16
GW = 80
NG = E // GW
GPT = NG // NS
IB = 10
NCH = GPT // IB
RPT = VP // NS
NSLOT = 3
ZR = NSLOT * GW


def _sc_body(xaug, idx2, out2, acc, ibuf, rows,
             gsem0, gsem1, ssem0, ssem1, isem):
    c = lax.axis_index("c")
    s = lax.axis_index("s")
    gsel = 1 - c
    base = s * RPT
    g0 = s * GPT
    gsems = (gsem0, gsem1)
    ssems = (ssem0, ssem1)

    def zrow(i, carry):
        for j in range(AUG // 16):
            rows[i, pl.ds(j * 16, 16)] = jnp.zeros((16,), jnp.float32)
        return carry

    lax.fori_loop(0, ZR, zrow, 0)
    pltpu.sync_copy(rows, acc.at[pl.ds(base, ZR)])
    pltpu.sync_copy(rows, acc.at[pl.ds(base + ZR, ZR)])
    pltpu.sync_copy(rows.at[pl.ds(0, RPT - 2 * ZR)],
                    acc.at[pl.ds(base + 2 * ZR, RPT - 2 * ZR)])
    plsc.subcore_barrier()

    def locate(g):
        ci = g // IB
        return lax.rem(ci, 2), g - ci * IB, lax.rem(g, NSLOT) * GW

    def fire_gather(g, parity):
        buf, j, slot = locate(g)
        pltpu.async_copy(xaug.at[ibuf.at[buf, j, gsel]],
                         rows.at[pl.ds(slot, GW)], gsems[parity])

    def wait_gather(g, parity):
        buf, j, slot = locate(g)
        pltpu.make_async_copy(xaug.at[ibuf.at[buf, j, gsel]],
                              rows.at[pl.ds(slot, GW)], gsems[parity]).wait()

    def fire_scatter(g, parity):
        buf, j, slot = locate(g)
        pltpu.async_copy(rows.at[pl.ds(slot, GW)],
                         acc.at[ibuf.at[buf, j, c]], ssems[parity], add=True)

    def wait_scatter(g, parity):
        buf, j, slot = locate(g)
        pltpu.make_async_copy(rows.at[pl.ds(slot, GW)],
                              acc.at[ibuf.at[buf, j, c]],
                              ssems[parity]).wait()

    pltpu.sync_copy(idx2.at[pl.ds(g0, IB)], ibuf.at[0])
    fire_gather(0, 0)
    fire_gather(1, 1)

    def step(b, carry):
        for p in range(2):
            g = 2 * b + p
            wait_gather(g, p)
            fire_scatter(g, p)

            @pl.when(g >= 1)
            def _():
                wait_scatter(g - 1, 1 - p)

            ci = g // IB
            j = g - ci * IB

            @pl.when(jnp.logical_and(j == 0, ci + 1 < NCH))
            def _():
                pltpu.async_copy(idx2.at[pl.ds(g0 + (ci + 1) * IB, IB)],
                                 ibuf.at[1 - lax.rem(ci, 2)], isem)

            @pl.when(g + 2 < GPT)
            def _():
                g2 = g + 2
                ci2 = g2 // IB

                @pl.when(g2 - ci2 * IB == 0)
                def _():
                    pltpu.make_async_copy(
                        idx2.at[pl.ds(g0 + ci2 * IB, IB)],
                        ibuf.at[lax.rem(ci2, 2)], isem).wait()

                fire_gather(g2, p)
        return carry

    lax.fori_loop(0, GPT // 2, step, 0)
    wait_scatter(GPT - 1, (GPT - 1) % 2)
    plsc.subcore_barrier()
    pltpu.sync_copy(acc.at[pl.ds(base, RPT)],
                    out2.at[c, pl.ds(base, RPT)])


_sc_accumulate = functools.partial(
    pl.kernel,
    out_type=jax.ShapeDtypeStruct((2, VP, AUG), jnp.float32),
    mesh=plsc.VectorSubcoreMesh(core_axis_name="c", subcore_axis_name="s"),
    compiler_params=pltpu.CompilerParams(use_tc_tiling_on_sc=False),
    scratch_types=[
        pltpu.VMEM_SHARED((VP, AUG), jnp.float32),
        pltpu.VMEM((2, IB, 2, GW), jnp.int32),
        pltpu.VMEM((ZR, AUG), jnp.float32),
        pltpu.SemaphoreType.DMA,
        pltpu.SemaphoreType.DMA,
        pltpu.SemaphoreType.DMA,
        pltpu.SemaphoreType.DMA,
        pltpu.SemaphoreType.DMA,
    ],
)(_sc_body)


def _mm_body(x_ref, p_ref, q_ref, w_ref, o_ref):
    xb = x_ref[...]
    pb = p_ref[0]
    qb = q_ref[0]
    w = w_ref[...]
    dout = pb[:, H:H + 1]
    din = qb[:, H:H + 1]
    dn = (((1,), (1,)), ((), ()))
    o_ref[...] = (
        lax.dot_general(xb * dout, w[:H, :H], dn, preferred_element_type=jnp.float32)
        + lax.dot_general(pb[:, :H], w[:H, H:], dn, preferred_element_type=jnp.float32)
        + lax.dot_general(qb[:, :H], w[H:, :H], dn, preferred_element_type=jnp.float32)
        + lax.dot_general(xb * din, w[H:, H:], dn, preferred_element_type=jnp.float32)
    )


_BM = 1000
_mm = pl.pallas_call(
    _mm_body,
    grid=(V // _BM,),
    in_specs=[
        pl.BlockSpec((_BM, H), lambda i: (i, 0)),
        pl.BlockSpec((1, _BM, AUG), lambda i: (0, i, 0)),
        pl.BlockSpec((1, _BM, AUG), lambda i: (1, i, 0)),
        pl.BlockSpec((2 * H, 2 * H), lambda i: (0, 0)),
    ],
    out_specs=pl.BlockSpec((_BM, H), lambda i: (i, 0)),
    out_shape=jax.ShapeDtypeStruct((V, H), jnp.float32),
)


def kernel(x, edge_index, W):
    idx2 = edge_index.reshape(2, NG, GW).transpose(1, 0, 2)
    xaug = jnp.concatenate(
        [x, jnp.ones((V, 1), jnp.float32), jnp.zeros((V, AUG - H - 1), jnp.float32)],
        axis=1,
    )
    pq = _sc_accumulate(xaug, idx2)
    return _mm(x, pq, pq, W)


# speedup vs baseline: 14.2208x; 1.1067x over previous
import functools

import jax
import jax.numpy as jnp
from jax import lax
from jax.experimental import pallas as pl
from jax.experimental.pallas import tpu as pltpu
from jax.experimental.pallas import tpu_sc as plsc

H = 128
V = 10000
VP = 10240
E = 320000
AUG = 144
NS = 16
GW = 80
NG = E // GW
GPT = NG // NS
IB = 10
NCH = GPT // IB
RPT = VP // NS
NSLOT = 3
ZR = NSLOT * GW


def _sc_body(xaug, idxr, out2, acc, ibuf, rows,
             gsem0, gsem1, ssem0, ssem1, isem):
    c = lax.axis_index("c")
    s = lax.axis_index("s")
    gsel = 1 - c
    base = s * RPT
    g0 = s * GPT
    gsems = (gsem0, gsem1)
    ssems = (ssem0, ssem1)

    def zrow(i, carry):
        for j in range(AUG // 16):
            rows[i, pl.ds(j * 16, 16)] = jnp.zeros((16,), jnp.float32)
        return carry

    lax.fori_loop(0, ZR, zrow, 0)
    pltpu.sync_copy(rows, acc.at[pl.ds(base, ZR)])
    pltpu.sync_copy(rows, acc.at[pl.ds(base + ZR, ZR)])
    pltpu.sync_copy(rows.at[pl.ds(0, RPT - 2 * ZR)],
                    acc.at[pl.ds(base + 2 * ZR, RPT - 2 * ZR)])
    plsc.subcore_barrier()

    def locate(g):
        ci = g // IB
        return lax.rem(ci, 2), g - ci * IB, lax.rem(g, NSLOT) * GW

    def load_chunk(ci, buf, fire_only):
        for e in range(2):
            d = pltpu.make_async_copy(
                idxr.at[e, pl.ds(g0 + ci * IB, IB)], ibuf.at[buf, e], isem)
            if fire_only:
                d.start()
            else:
                d.wait()

    def fire_gather(g, parity):
        buf, j, slot = locate(g)
        pltpu.async_copy(xaug.at[ibuf.at[buf, gsel, j]],
                         rows.at[pl.ds(slot, GW)], gsems[parity])

    def wait_gather(g, parity):
        buf, j, slot = locate(g)
        pltpu.make_async_copy(xaug.at[ibuf.at[buf, gsel, j]],
                              rows.at[pl.ds(slot, GW)], gsems[parity]).wait()

    def fire_scatter(g, parity):
        buf, j, slot = locate(g)
        pltpu.async_copy(rows.at[pl.ds(slot, GW)],
                         acc.at[ibuf.at[buf, c, j]], ssems[parity], add=True)

    def wait_scatter(g, parity):
        buf, j, slot = locate(g)
        pltpu.make_async_copy(rows.at[pl.ds(slot, GW)],
                              acc.at[ibuf.at[buf, c, j]],
                              ssems[parity]).wait()

    load_chunk(0, 0, fire_only=True)
    load_chunk(0, 0, fire_only=False)
    fire_gather(0, 0)
    fire_gather(1, 1)

    def step(b, carry):
        for p in range(2):
            g = 2 * b + p
            wait_gather(g, p)
            fire_scatter(g, p)

            @pl.when(g >= 1)
            def _():
                wait_scatter(g - 1, 1 - p)

            ci = g // IB
            j = g - ci * IB

            @pl.when(jnp.logical_and(j == 0, ci + 1 < NCH))
            def _():
                load_chunk(ci + 1, 1 - lax.rem(ci, 2), fire_only=True)

            @pl.when(g + 2 < GPT)
            def _():
                g2 = g + 2
                ci2 = g2 // IB

                @pl.when(g2 - ci2 * IB == 0)
                def _():
                    load_chunk(ci2, lax.rem(ci2, 2), fire_only=False)

                fire_gather(g2, p)
        return carry

    lax.fori_loop(0, GPT // 2, step, 0)
    wait_scatter(GPT - 1, (GPT - 1) % 2)
    plsc.subcore_barrier()
    pltpu.sync_copy(acc.at[pl.ds(base, RPT)],
                    out2.at[c, pl.ds(base, RPT)])


_sc_accumulate = functools.partial(
    pl.kernel,
    out_type=jax.ShapeDtypeStruct((2, VP, AUG), jnp.float32),
    mesh=plsc.VectorSubcoreMesh(core_axis_name="c", subcore_axis_name="s"),
    compiler_params=pltpu.CompilerParams(use_tc_tiling_on_sc=False),
    scratch_types=[
        pltpu.VMEM_SHARED((VP, AUG), jnp.float32),
        pltpu.VMEM((2, 2, IB, GW), jnp.int32),
        pltpu.VMEM((ZR, AUG), jnp.float32),
        pltpu.SemaphoreType.DMA,
        pltpu.SemaphoreType.DMA,
        pltpu.SemaphoreType.DMA,
        pltpu.SemaphoreType.DMA,
        pltpu.SemaphoreType.DMA,
    ],
)(_sc_body)


def _mm_body(x_ref, p_ref, q_ref, w_ref, o_ref):
    xb = x_ref[...]
    pb = p_ref[0]
    qb = q_ref[0]
    w = w_ref[...]
    dout = pb[:, H:H + 1]
    din = qb[:, H:H + 1]
    dn = (((1,), (1,)), ((), ()))
    o_ref[...] = (
        lax.dot_general(xb * dout, w[:H, :H], dn, preferred_element_type=jnp.float32)
        + lax.dot_general(pb[:, :H], w[:H, H:], dn, preferred_element_type=jnp.float32)
        + lax.dot_general(qb[:, :H], w[H:, :H], dn, preferred_element_type=jnp.float32)
        + lax.dot_general(xb * din, w[H:, H:], dn, preferred_element_type=jnp.float32)
    )


_BM = 1000
_mm = pl.pallas_call(
    _mm_body,
    grid=(V // _BM,),
    in_specs=[
        pl.BlockSpec((_BM, H), lambda i: (i, 0)),
        pl.BlockSpec((1, _BM, AUG), lambda i: (0, i, 0)),
        pl.BlockSpec((1, _BM, AUG), lambda i: (1, i, 0)),
        pl.BlockSpec((2 * H, 2 * H), lambda i: (0, 0)),
    ],
    out_specs=pl.BlockSpec((_BM, H), lambda i: (i, 0)),
    out_shape=jax.ShapeDtypeStruct((V, H), jnp.float32),
)


def kernel(x, edge_index, W):
    idxr = edge_index.reshape(2, NG, GW)
    xaug = jnp.concatenate(
        [x, jnp.ones((V, 1), jnp.float32), jnp.zeros((V, AUG - H - 1), jnp.float32)],
        axis=1,
    )
    pq = _sc_accumulate(xaug, idxr)
    return _mm(x, pq, pq, W)


# speedup vs baseline: 16.2293x; 1.1412x over previous
import functools

import jax
import jax.numpy as jnp
from jax import lax
from jax.experimental import pallas as pl
from jax.experimental.pallas import tpu as pltpu
from jax.experimental.pallas import tpu_sc as plsc

H = 128
V = 10000
VP = 10240
E = 320000
NS = 16
GW = 80
NG = E // GW
GPT = NG // NS
IB = 10
NCH = GPT // IB
RPT = VP // NS
NSLOT = 3
ZR = NSLOT * GW
DR = VP // H
DRT = DR // NS


def _sc_body(x, idxr, out_pq, out_sx, acc, deg_sh, ibuf, rows, hist, iidx,
             degv, gsem0, gsem1, ssem0, ssem1, isem):
    c = lax.axis_index("c")
    s = lax.axis_index("s")
    gsel = 1 - c
    base = s * RPT
    g0 = s * GPT
    gsems = (gsem0, gsem1)
    ssems = (ssem0, ssem1)
    ones16 = jnp.ones((16,), jnp.float32)

    def zrow(i, carry):
        for j in range(H // 16):
            rows[i, pl.ds(j * 16, 16)] = jnp.zeros((16,), jnp.float32)
        return carry

    lax.fori_loop(0, ZR, zrow, 0)

    def hrow(i, carry):
        for j in range(H // 16):
            hist[i, pl.ds(j * 16, 16)] = jnp.zeros((16,), jnp.float32)
        return carry

    lax.fori_loop(0, DR, hrow, 0)
    for k in range(DR // 16):
        iidx[0, pl.ds(k * 16, 16)] = lax.iota(jnp.int32, 16) + (k * 16)

    pltpu.sync_copy(rows, acc.at[pl.ds(base, ZR)])
    pltpu.sync_copy(rows, acc.at[pl.ds(base + ZR, ZR)])
    pltpu.sync_copy(rows.at[pl.ds(0, RPT - 2 * ZR)],
                    acc.at[pl.ds(base + 2 * ZR, RPT - 2 * ZR)])
    pltpu.sync_copy(rows.at[pl.ds(0, DRT)], deg_sh.at[pl.ds(s * DRT, DRT)])
    plsc.subcore_barrier()

    def locate(g):
        ci = g // IB
        return lax.rem(ci, 2), g - ci * IB, lax.rem(g, NSLOT) * GW

    def load_chunk(ci, buf, fire_only):
        for e in range(2):
            d = pltpu.make_async_copy(
                idxr.at[e, pl.ds(g0 + ci * IB, IB)], ibuf.at[buf, e], isem)
            if fire_only:
                d.start()
            else:
                d.wait()

    def fire_gather(g, parity):
        buf, j, slot = locate(g)
        pltpu.async_copy(x.at[ibuf.at[buf, gsel, j]],
                         rows.at[pl.ds(slot, GW)], gsems[parity])

    def wait_gather(g, parity):
        buf, j, slot = locate(g)
        pltpu.make_async_copy(x.at[ibuf.at[buf, gsel, j]],
                              rows.at[pl.ds(slot, GW)], gsems[parity]).wait()

    def fire_scatter(g, parity):
        buf, j, slot = locate(g)
        pltpu.async_copy(rows.at[pl.ds(slot, GW)],
                         acc.at[ibuf.at[buf, c, j]], ssems[parity], add=True)

    def wait_scatter(g, parity):
        buf, j, slot = locate(g)
        pltpu.make_async_copy(rows.at[pl.ds(slot, GW)],
                              acc.at[ibuf.at[buf, c, j]],
                              ssems[parity]).wait()

    def hist_update(g):
        buf, j, _ = locate(g)
        for k in range(GW // 16):
            v = ibuf[buf, c, j, pl.ds(k * 16, 16)]
            plsc.addupdate_scatter(
                hist,
                [lax.shift_right_logical(v, 7), jnp.bitwise_and(v, 127)],
                ones16)

    load_chunk(0, 0, fire_only=True)
    load_chunk(0, 0, fire_only=False)
    fire_gather(0, 0)
    fire_gather(1, 1)

    def step(b, carry):
        for p in range(2):
            g = 2 * b + p
            wait_gather(g, p)
            fire_scatter(g, p)
            hist_update(g)

            @pl.when(g >= 1)
            def _():
                wait_scatter(g - 1, 1 - p)

            ci = g // IB
            j = g - ci * IB

            @pl.when(jnp.logical_and(j == 0, ci + 1 < NCH))
            def _():
                load_chunk(ci + 1, 1 - lax.rem(ci, 2), fire_only=True)

            @pl.when(g + 2 < GPT)
            def _():
                g2 = g + 2
                ci2 = g2 // IB

                @pl.when(g2 - ci2 * IB == 0)
                def _():
                    load_chunk(ci2, lax.rem(ci2, 2), fire_only=False)

                fire_gather(g2, p)
        return carry

    lax.fori_loop(0, GPT // 2, step, 0)
    wait_scatter(GPT - 1, (GPT - 1) % 2)
    pltpu.sync_copy(hist, deg_sh.at[iidx.at[0]], add=True)
    plsc.subcore_barrier()

    pltpu.sync_copy(acc.at[pl.ds(base, RPT)], out_pq.at[c, pl.ds(base, RPT)])
    for k in range(RPT // H):
        n0 = base + k * H
        pltpu.sync_copy(x.at[pl.ds(n0, H)], rows.at[pl.ds(0, H)])
        pltpu.sync_copy(deg_sh.at[pl.ds(s * DRT + k, 1)], degv)

        def scale(r, carry):
            d = plsc.load_gather(
                degv, [jnp.zeros((16,), jnp.int32),
                       jnp.full((16,), r, jnp.int32)])
            for q in range(H // 16):
                rows[r, pl.ds(q * 16, 16)] = rows[r, pl.ds(q * 16, 16)] * d
            return carry

        lax.fori_loop(0, H, scale, 0)
        pltpu.sync_copy(rows.at[pl.ds(0, H)], out_sx.at[c, pl.ds(n0, H)])


_sc_accumulate = functools.partial(
    pl.kernel,
    out_type=(
        jax.ShapeDtypeStruct((2, VP, H), jnp.float32),
        jax.ShapeDtypeStruct((2, VP, H), jnp.float32),
    ),
    mesh=plsc.VectorSubcoreMesh(core_axis_name="c", subcore_axis_name="s"),
    compiler_params=pltpu.CompilerParams(use_tc_tiling_on_sc=False,
                                         needs_layout_passes=False),
    scratch_types=[
        pltpu.VMEM_SHARED((VP, H), jnp.float32),
        pltpu.VMEM_SHARED((DR, H), jnp.float32),
        pltpu.VMEM((2, 2, IB, GW), jnp.int32),
        pltpu.VMEM((ZR, H), jnp.float32),
        pltpu.VMEM((DR, H), jnp.float32),
        pltpu.VMEM((1, DR), jnp.int32),
        pltpu.VMEM((1, H), jnp.float32),
        pltpu.SemaphoreType.DMA,
        pltpu.SemaphoreType.DMA,
        pltpu.SemaphoreType.DMA,
        pltpu.SemaphoreType.DMA,
        pltpu.SemaphoreType.DMA,
    ],
)(_sc_body)


def _mm_body(p_ref, q_ref, sp_ref, sq_ref, w_ref, o_ref):
    pb = p_ref[0]
    qb = q_ref[0]
    spb = sp_ref[0]
    sqb = sq_ref[0]
    w = w_ref[...]
    dn = (((1,), (1,)), ((), ()))
    o_ref[...] = (
        lax.dot_general(spb, w[:H, :H], dn, preferred_element_type=jnp.float32)
        + lax.dot_general(pb, w[:H, H:], dn, preferred_element_type=jnp.float32)
        + lax.dot_general(qb, w[H:, :H], dn, preferred_element_type=jnp.float32)
        + lax.dot_general(sqb, w[H:, H:], dn, preferred_element_type=jnp.float32)
    )


_BM = 1000
_mm = pl.pallas_call(
    _mm_body,
    grid=(V // _BM,),
    in_specs=[
        pl.BlockSpec((1, _BM, H), lambda i: (0, i, 0)),
        pl.BlockSpec((1, _BM, H), lambda i: (1, i, 0)),
        pl.BlockSpec((1, _BM, H), lambda i: (0, i, 0)),
        pl.BlockSpec((1, _BM, H), lambda i: (1, i, 0)),
        pl.BlockSpec((2 * H, 2 * H), lambda i: (0, 0)),
    ],
    out_specs=pl.BlockSpec((_BM, H), lambda i: (i, 0)),
    out_shape=jax.ShapeDtypeStruct((V, H), jnp.float32),
)


def kernel(x, edge_index, W):
    idxr = edge_index.reshape(2, NG, GW)
    pq, sx = _sc_accumulate(x, idxr)
    return _mm(pq, pq, sx, sx, W)


# speedup vs baseline: 16.6171x; 1.0239x over previous
import functools

import jax
import jax.numpy as jnp
from jax import lax
from jax.experimental import pallas as pl
from jax.experimental.pallas import tpu as pltpu
from jax.experimental.pallas import tpu_sc as plsc

H = 128
V = 10000
VP = 10240
E = 320000
NS = 16
GW = 80
NG = E // GW
GPT = NG // NS
IB = 10
NCH = GPT // IB
CHW = IB * GW
RPT = VP // NS
SXC = 64
NSLOT = 3
ZR = NSLOT * GW
DR = VP // H
DRT = DR // NS


def _sc_body(x, idxr, out_pq, out_sx, acc, deg_sh, ibuf, rows, hist, iidx,
             degv, gsem0, gsem1, ssem0, ssem1, isem):
    c = lax.axis_index("c")
    s = lax.axis_index("s")
    gsel = 1 - c
    base = s * RPT
    crow = s * NCH
    gsems = (gsem0, gsem1)
    ssems = (ssem0, ssem1)
    ones16 = jnp.ones((16,), jnp.float32)

    def zrow(i, carry):
        for j in range(H // 16):
            rows[i, pl.ds(j * 16, 16)] = jnp.zeros((16,), jnp.float32)
        return carry

    lax.fori_loop(0, ZR, zrow, 0)

    def hrow(i, carry):
        for j in range(H // 16):
            hist[i, pl.ds(j * 16, 16)] = jnp.zeros((16,), jnp.float32)
        return carry

    lax.fori_loop(0, DR, hrow, 0)
    for k in range(DR // 16):
        iidx[0, pl.ds(k * 16, 16)] = lax.iota(jnp.int32, 16) + (k * 16)

    pltpu.sync_copy(rows, acc.at[pl.ds(base, ZR)])
    pltpu.sync_copy(rows, acc.at[pl.ds(base + ZR, ZR)])
    pltpu.sync_copy(rows.at[pl.ds(0, RPT - 2 * ZR)],
                    acc.at[pl.ds(base + 2 * ZR, RPT - 2 * ZR)])
    pltpu.sync_copy(rows.at[pl.ds(0, DRT)], deg_sh.at[pl.ds(s * DRT, DRT)])
    plsc.subcore_barrier()

    def locate(g):
        ci = g // IB
        return lax.rem(ci, 2), g - ci * IB, lax.rem(g, NSLOT) * GW

    def load_chunk(ci, buf, fire_only):
        for e in range(2):
            d = pltpu.make_async_copy(
                idxr.at[e, crow + ci], ibuf.at[buf, e], isem)
            if fire_only:
                d.start()
            else:
                d.wait()

    def fire_gather(g, parity):
        buf, j, slot = locate(g)
        pltpu.async_copy(x.at[ibuf.at[buf, gsel, pl.ds(j * GW, GW)]],
                         rows.at[pl.ds(slot, GW)], gsems[parity])

    def wait_gather(g, parity):
        buf, j, slot = locate(g)
        pltpu.make_async_copy(x.at[ibuf.at[buf, gsel, pl.ds(j * GW, GW)]],
                              rows.at[pl.ds(slot, GW)], gsems[parity]).wait()

    def fire_scatter(g, parity):
        buf, j, slot = locate(g)
        pltpu.async_copy(rows.at[pl.ds(slot, GW)],
                         acc.at[ibuf.at[buf, c, pl.ds(j * GW, GW)]],
                         ssems[parity], add=True)

    def wait_scatter(g, parity):
        buf, j, slot = locate(g)
        pltpu.make_async_copy(rows.at[pl.ds(slot, GW)],
                              acc.at[ibuf.at[buf, c, pl.ds(j * GW, GW)]],
                              ssems[parity]).wait()

    def hist_update(g):
        buf, j, _ = locate(g)
        for k in range(GW // 16):
            v = ibuf[buf, c, pl.ds(j * GW + k * 16, 16)]
            plsc.addupdate_scatter(
                hist,
                [lax.shift_right_logical(v, 7), jnp.bitwise_and(v, 127)],
                ones16)

    load_chunk(0, 0, fire_only=True)
    load_chunk(0, 0, fire_only=False)
    fire_gather(0, 0)
    fire_gather(1, 1)

    def step(b, carry):
        for p in range(2):
            g = 2 * b + p
            wait_gather(g, p)
            fire_scatter(g, p)
            hist_update(g)

            @pl.when(g >= 1)
            def _():
                wait_scatter(g - 1, 1 - p)

            ci = g // IB
            j = g - ci * IB

            @pl.when(jnp.logical_and(j == 0, ci + 1 < NCH))
            def _():
                load_chunk(ci + 1, 1 - lax.rem(ci, 2), fire_only=True)

            @pl.when(g + 2 < GPT)
            def _():
                g2 = g + 2
                ci2 = g2 // IB

                @pl.when(g2 - ci2 * IB == 0)
                def _():
                    load_chunk(ci2, lax.rem(ci2, 2), fire_only=False)

                fire_gather(g2, p)
        return carry

    lax.fori_loop(0, GPT // 2, step, 0)
    wait_scatter(GPT - 1, (GPT - 1) % 2)
    pltpu.sync_copy(hist, deg_sh.at[iidx.at[0]], add=True)
    plsc.subcore_barrier()

    pq_desc = pltpu.make_async_copy(
        acc.at[pl.ds(base, RPT)], out_pq.at[c, pl.ds(base, RPT)], isem)
    pq_desc.start()
    NSX = RPT // SXC

    def x_desc(k):
        slot = (k % 2) * SXC
        return pltpu.make_async_copy(
            x.at[pl.ds(base + k * SXC, SXC)], rows.at[pl.ds(slot, SXC)],
            gsems[k % 2])

    def w_desc(k):
        slot = (k % 2) * SXC
        return pltpu.make_async_copy(
            rows.at[pl.ds(slot, SXC)],
            out_sx.at[c, pl.ds(base + k * SXC, SXC)], ssems[k % 2])

    x_desc(0).start()
    x_desc(1).start()
    for k in range(NSX):
        slot = (k % 2) * SXC
        if k % 2 == 0:
            pltpu.sync_copy(deg_sh.at[pl.ds(s * DRT + k // 2, 1)], degv)
        x_desc(k).wait()

        def scale(r, carry):
            d = plsc.load_gather(
                degv, [jnp.zeros((16,), jnp.int32),
                       jnp.full((16,), (k % 2) * SXC + r, jnp.int32)])
            for q in range(H // 16):
                rows[slot + r, pl.ds(q * 16, 16)] = (
                    rows[slot + r, pl.ds(q * 16, 16)] * d)
            return carry

        lax.fori_loop(0, SXC, scale, 0)
        w_desc(k).start()
        if k >= 1:
            w_desc(k - 1).wait()
            if k + 1 < NSX:
                x_desc(k + 1).start()
    w_desc(NSX - 1).wait()
    pq_desc.wait()


_sc_accumulate = functools.partial(
    pl.kernel,
    out_type=(
        jax.ShapeDtypeStruct((2, VP, H), jnp.float32),
        jax.ShapeDtypeStruct((2, VP, H), jnp.float32),
    ),
    mesh=plsc.VectorSubcoreMesh(core_axis_name="c", subcore_axis_name="s"),
    compiler_params=pltpu.CompilerParams(use_tc_tiling_on_sc=False,
                                         needs_layout_passes=False),
    scratch_types=[
        pltpu.VMEM_SHARED((VP, H), jnp.float32),
        pltpu.VMEM_SHARED((DR, H), jnp.float32),
        pltpu.VMEM((2, 2, CHW), jnp.int32),
        pltpu.VMEM((ZR, H), jnp.float32),
        pltpu.VMEM((DR, H), jnp.float32),
        pltpu.VMEM((1, DR), jnp.int32),
        pltpu.VMEM((1, H), jnp.float32),
        pltpu.SemaphoreType.DMA,
        pltpu.SemaphoreType.DMA,
        pltpu.SemaphoreType.DMA,
        pltpu.SemaphoreType.DMA,
        pltpu.SemaphoreType.DMA,
    ],
)(_sc_body)


def _mm_body(p_ref, q_ref, sp_ref, sq_ref, w_ref, o_ref):
    hcat = jnp.concatenate(
        [sp_ref[0], p_ref[0], q_ref[0], sq_ref[0]], axis=1)
    w = w_ref[...]
    wcat = jnp.concatenate([w[:H, :], w[H:, :]], axis=1)
    o_ref[...] = lax.dot_general(
        hcat, wcat, (((1,), (1,)), ((), ())),
        preferred_element_type=jnp.float32)


_BM = 1000
_mm = pl.pallas_call(
    _mm_body,
    grid=(V // _BM,),
    in_specs=[
        pl.BlockSpec((1, _BM, H), lambda i: (0, i, 0)),
        pl.BlockSpec((1, _BM, H), lambda i: (1, i, 0)),
        pl.BlockSpec((1, _BM, H), lambda i: (0, i, 0)),
        pl.BlockSpec((1, _BM, H), lambda i: (1, i, 0)),
        pl.BlockSpec((2 * H, 2 * H), lambda i: (0, 0)),
    ],
    out_specs=pl.BlockSpec((_BM, H), lambda i: (i, 0)),
    out_shape=jax.ShapeDtypeStruct((V, H), jnp.float32),
)


def kernel(x, edge_index, W):
    idxr = edge_index.reshape(2, NS * NCH, CHW)
    pq, sx = _sc_accumulate(x, idxr)
    return _mm(pq, pq, sx, sx, W)


# speedup vs baseline: 17.5774x; 1.0578x over previous
import functools

import jax
import jax.numpy as jnp
from jax import lax
from jax.experimental import pallas as pl
from jax.experimental.pallas import tpu as pltpu
from jax.experimental.pallas import tpu_sc as plsc

H = 128
V = 10000
VP = 10240
E = 320000
NS = 16
GW = 80
NG = E // GW
GPT = NG // NS
IB = 10
NCH = GPT // IB
CHW = IB * GW
RPT = VP // NS
SXC = 64
NSLOT = 3
ZR = NSLOT * GW
DR = VP // H
DRT = DR // NS


def _sc_body(x, idxr, out_pq, out_deg, acc, deg_sh, ibuf, rows, hist, iidx,
             gsem0, gsem1, ssem0, ssem1, isem):
    c = lax.axis_index("c")
    s = lax.axis_index("s")
    gsel = 1 - c
    base = s * RPT
    crow = s * NCH
    gsems = (gsem0, gsem1)
    ssems = (ssem0, ssem1)
    ones16 = jnp.ones((16,), jnp.float32)

    def zrow(i, carry):
        for j in range(H // 16):
            rows[i, pl.ds(j * 16, 16)] = jnp.zeros((16,), jnp.float32)
        return carry

    lax.fori_loop(0, ZR, zrow, 0)

    def hrow(i, carry):
        for j in range(H // 16):
            hist[i, pl.ds(j * 16, 16)] = jnp.zeros((16,), jnp.float32)
        return carry

    lax.fori_loop(0, DR, hrow, 0)
    for k in range(DR // 16):
        iidx[0, pl.ds(k * 16, 16)] = lax.iota(jnp.int32, 16) + (k * 16)

    pltpu.sync_copy(rows, acc.at[pl.ds(base, ZR)])
    pltpu.sync_copy(rows, acc.at[pl.ds(base + ZR, ZR)])
    pltpu.sync_copy(rows.at[pl.ds(0, RPT - 2 * ZR)],
                    acc.at[pl.ds(base + 2 * ZR, RPT - 2 * ZR)])
    pltpu.sync_copy(rows.at[pl.ds(0, DRT)], deg_sh.at[pl.ds(s * DRT, DRT)])
    plsc.subcore_barrier()

    def locate(g):
        ci = g // IB
        return lax.rem(ci, 2), g - ci * IB, lax.rem(g, NSLOT) * GW

    def load_chunk(ci, buf, fire_only):
        for e in range(2):
            d = pltpu.make_async_copy(
                idxr.at[e, crow + ci], ibuf.at[buf, e], isem)
            if fire_only:
                d.start()
            else:
                d.wait()

    def fire_gather(g, parity):
        buf, j, slot = locate(g)
        pltpu.async_copy(x.at[ibuf.at[buf, gsel, pl.ds(j * GW, GW)]],
                         rows.at[pl.ds(slot, GW)], gsems[parity])

    def wait_gather(g, parity):
        buf, j, slot = locate(g)
        pltpu.make_async_copy(x.at[ibuf.at[buf, gsel, pl.ds(j * GW, GW)]],
                              rows.at[pl.ds(slot, GW)], gsems[parity]).wait()

    def fire_scatter(g, parity):
        buf, j, slot = locate(g)
        pltpu.async_copy(rows.at[pl.ds(slot, GW)],
                         acc.at[ibuf.at[buf, c, pl.ds(j * GW, GW)]],
                         ssems[parity], add=True)

    def wait_scatter(g, parity):
        buf, j, slot = locate(g)
        pltpu.make_async_copy(rows.at[pl.ds(slot, GW)],
                              acc.at[ibuf.at[buf, c, pl.ds(j * GW, GW)]],
                              ssems[parity]).wait()

    def hist_update(g):
        buf, j, _ = locate(g)
        for k in range(GW // 16):
            v = ibuf[buf, c, pl.ds(j * GW + k * 16, 16)]
            plsc.addupdate_scatter(
                hist,
                [lax.shift_right_logical(v, 7), jnp.bitwise_and(v, 127)],
                ones16)

    load_chunk(0, 0, fire_only=True)
    load_chunk(0, 0, fire_only=False)
    fire_gather(0, 0)
    fire_gather(1, 1)

    def step(b, carry):
        for p in range(2):
            g = 2 * b + p
            wait_gather(g, p)
            fire_scatter(g, p)
            hist_update(g)

            @pl.when(g >= 1)
            def _():
                wait_scatter(g - 1, 1 - p)

            ci = g // IB
            j = g - ci * IB

            @pl.when(jnp.logical_and(j == 0, ci + 1 < NCH))
            def _():
                load_chunk(ci + 1, 1 - lax.rem(ci, 2), fire_only=True)

            @pl.when(g + 2 < GPT)
            def _():
                g2 = g + 2
                ci2 = g2 // IB

                @pl.when(g2 - ci2 * IB == 0)
                def _():
                    load_chunk(ci2, lax.rem(ci2, 2), fire_only=False)

                fire_gather(g2, p)
        return carry

    lax.fori_loop(0, GPT // 2, step, 0)
    wait_scatter(GPT - 1, (GPT - 1) % 2)
    pltpu.sync_copy(hist, deg_sh.at[iidx.at[0]], add=True)
    plsc.subcore_barrier()

    pq_desc = pltpu.make_async_copy(
        acc.at[pl.ds(base, RPT)], out_pq.at[c, pl.ds(base, RPT)], isem)
    pq_desc.start()
    pltpu.sync_copy(deg_sh.at[pl.ds(s * DRT, DRT)],
                    out_deg.at[c, pl.ds(s * DRT, DRT)])
    pq_desc.wait()


_sc_accumulate = functools.partial(
    pl.kernel,
    out_type=(
        jax.ShapeDtypeStruct((2, VP, H), jnp.float32),
        jax.ShapeDtypeStruct((2, DR, H), jnp.float32),
    ),
    mesh=plsc.VectorSubcoreMesh(core_axis_name="c", subcore_axis_name="s"),
    compiler_params=pltpu.CompilerParams(use_tc_tiling_on_sc=False,
                                         needs_layout_passes=False),
    scratch_types=[
        pltpu.VMEM_SHARED((VP, H), jnp.float32),
        pltpu.VMEM_SHARED((DR, H), jnp.float32),
        pltpu.VMEM((2, 2, CHW), jnp.int32),
        pltpu.VMEM((ZR, H), jnp.float32),
        pltpu.VMEM((DR, H), jnp.float32),
        pltpu.VMEM((1, DR), jnp.int32),
        pltpu.SemaphoreType.DMA,
        pltpu.SemaphoreType.DMA,
        pltpu.SemaphoreType.DMA,
        pltpu.SemaphoreType.DMA,
        pltpu.SemaphoreType.DMA,
    ],
)(_sc_body)


def _mm_body(d_ref, x_ref, p_ref, q_ref, w_ref, o_ref):
    xb = x_ref[...]
    sel = jnp.where(
        lax.broadcasted_iota(jnp.int32, (_BM, _DB), 0) // H
        == lax.broadcasted_iota(jnp.int32, (_BM, _DB), 1),
        1.0, 0.0)
    lane_pick = jnp.where(
        lax.broadcasted_iota(jnp.int32, (_BM, H), 1)
        == lax.rem(lax.broadcasted_iota(jnp.int32, (_BM, H), 0), H),
        1.0, 0.0)

    def spread(u):
        z = lax.dot_general(sel, u, (((1,), (0,)), ((), ())),
                            preferred_element_type=jnp.float32)
        return jnp.sum(z * lane_pick, axis=1, keepdims=True)

    dout = spread(d_ref[0, 0])
    din = spread(d_ref[1, 0])
    hcat = jnp.concatenate(
        [xb * dout, p_ref[0], q_ref[0], xb * din], axis=1)
    w = w_ref[...]
    wcat = jnp.concatenate([w[:H, :], w[H:, :]], axis=1)
    o_ref[...] = lax.dot_general(
        hcat, wcat, (((1,), (1,)), ((), ())),
        preferred_element_type=jnp.float32)


_BM = 1280
_DB = _BM // H
_mm = pl.pallas_call(
    _mm_body,
    grid=((V + _BM - 1) // _BM,),
    in_specs=[
        pl.BlockSpec((2, 1, _DB, H), lambda i: (0, i, 0, 0)),
        pl.BlockSpec((_BM, H), lambda i: (i, 0)),
        pl.BlockSpec((1, _BM, H), lambda i: (0, i, 0)),
        pl.BlockSpec((1, _BM, H), lambda i: (1, i, 0)),
        pl.BlockSpec((2 * H, 2 * H), lambda i: (0, 0)),
    ],
    out_specs=pl.BlockSpec((_BM, H), lambda i: (i, 0)),
    out_shape=jax.ShapeDtypeStruct((V, H), jnp.float32),
)


def kernel(x, edge_index, W):
    idxr = edge_index.reshape(2, NS * NCH, CHW)
    pq, deg = _sc_accumulate(x, idxr)
    deg4 = deg.reshape(2, DR // _DB, _DB, H)
    return _mm(deg4, x, pq, pq, W)
